# Initial kernel scaffold; baseline (speedup 1.0000x reference)
import jax
import jax.numpy as jnp
from jax.experimental import pallas as pl


def kernel(x, edge_index, batch, W_in, b_in, Ws, bs, gammas, betas, bn_means, bn_vars, W1, b1, W2, b2):
    raise NotImplementedError("write your pallas kernel here")


# speedup vs baseline: 8.0652x; 8.0652x over previous
import functools

import jax
import jax.numpy as jnp
from jax import lax
from jax.experimental import pallas as pl
from jax.experimental.pallas import tpu as pltpu
from jax.experimental.pallas import tpu_sc as plsc

N = 10000
E = 320000
H = 128
L = 4
C = 10
G = 64

NP = 10240
ROWS_PER_TILE = NP // 16
K = 128
EPW = 79 * K
EP = 32 * EPW
NCHUNK = EPW // K
R = 2048
GRID = NP // R


def _sc_mesh():
    return plsc.VectorSubcoreMesh(core_axis_name="c", subcore_axis_name="s")


def _deg_kernel(dst_hbm, ones_hbm, zeros_hbm, cnt_hbm, idx_v, ones_v, stage_v,
                acc_sh, sem):
    c = lax.axis_index("c")
    s = lax.axis_index("s")
    w = c * 16 + s

    pltpu.sync_copy(zeros_hbm, stage_v)
    pltpu.sync_copy(stage_v, acc_sh.at[pl.ds(s * ROWS_PER_TILE, ROWS_PER_TILE)])
    pltpu.sync_copy(ones_hbm, ones_v)
    plsc.subcore_barrier()

    def body(i, carry):
        base = w * EPW + i * K
        pltpu.sync_copy(dst_hbm.at[pl.ds(base, K)], idx_v)
        pltpu.sync_copy(ones_v, acc_sh.at[idx_v], add=True)
        return carry

    lax.fori_loop(0, NCHUNK, body, 0)
    plsc.subcore_barrier()

    pltpu.sync_copy(acc_sh.at[pl.ds(s * ROWS_PER_TILE, ROWS_PER_TILE)], stage_v)
    pltpu.sync_copy(stage_v, cnt_hbm.at[c, pl.ds(s * ROWS_PER_TILE, ROWS_PER_TILE)])


def _degrees(dst_pad, ones128, zeros640):
    k = functools.partial(
        pl.kernel,
        mesh=_sc_mesh(),
        out_type=jax.ShapeDtypeStruct((2, NP), jnp.float32),
        scratch_types=[
            pltpu.VMEM((K,), jnp.int32),
            pltpu.VMEM((K,), jnp.float32),
            pltpu.VMEM((ROWS_PER_TILE,), jnp.float32),
            pltpu.VMEM_SHARED((NP,), jnp.float32),
            pltpu.SemaphoreType.DMA,
        ],
    )(_deg_kernel)
    return k(dst_pad, ones128, zeros640)


def _mp_kernel(gs_hbm, src_hbm, dst_hbm, zeros_hbm, out_hbm, idxs_v, idxd_v,
               rows_v, stage_v, acc_sh, sem):
    c = lax.axis_index("c")
    s = lax.axis_index("s")
    w = c * 16 + s

    pltpu.sync_copy(zeros_hbm, stage_v)
    for t in range(ROWS_PER_TILE // K):
        pltpu.sync_copy(
            stage_v, acc_sh.at[pl.ds(s * ROWS_PER_TILE + t * K, K)])
    plsc.subcore_barrier()

    def body(i, carry):
        base = w * EPW + i * K
        pltpu.sync_copy(src_hbm.at[pl.ds(base, K)], idxs_v)
        pltpu.sync_copy(dst_hbm.at[pl.ds(base, K)], idxd_v)
        pltpu.async_copy(gs_hbm.at[idxs_v], rows_v, sem).wait()
        pltpu.sync_copy(rows_v, acc_sh.at[idxd_v], add=True)
        return carry

    lax.fori_loop(0, NCHUNK, body, 0)
    plsc.subcore_barrier()

    for t in range(ROWS_PER_TILE // K):
        r0 = s * ROWS_PER_TILE + t * K
        pltpu.sync_copy(acc_sh.at[pl.ds(r0, K)], rows_v)
        pltpu.sync_copy(rows_v, out_hbm.at[c, pl.ds(r0, K)])


def _message_pass(gs, src_pad, dst_pad, zeros_rows):
    k = functools.partial(
        pl.kernel,
        mesh=_sc_mesh(),
        out_type=jax.ShapeDtypeStruct((2, NP, H), jnp.float32),
        scratch_types=[
            pltpu.VMEM((K,), jnp.int32),
            pltpu.VMEM((K,), jnp.int32),
            pltpu.VMEM((K, H), jnp.float32),
            pltpu.VMEM((K, H), jnp.float32),
            pltpu.VMEM_SHARED((NP, H), jnp.float32),
            pltpu.SemaphoreType.DMA,
        ],
    )(_mp_kernel)
    return k(gs, src_pad, dst_pad, zeros_rows)


def _proj_body(x_ref, cnt_ref, Wi_ref, bi_ref, W0_ref, gs_ref, dinv_ref):
    cnt = cnt_ref[0] + cnt_ref[1] + 1.0
    dinv = lax.rsqrt(jnp.maximum(cnt, 1.0))
    h = jnp.maximum(
        jnp.dot(x_ref[...], Wi_ref[...], preferred_element_type=jnp.float32)
        + bi_ref[...], 0.0)
    gs_ref[...] = jnp.dot(h, W0_ref[...],
                          preferred_element_type=jnp.float32) * dinv
    dinv_ref[...] = dinv


def _proj(x_pad, cnt2, W_in, b_in, W0):
    return pl.pallas_call(
        _proj_body,
        grid=(GRID,),
        in_specs=[
            pl.BlockSpec((R, H), lambda j: (j, 0)),
            pl.BlockSpec((2, R, 1), lambda j: (0, j, 0)),
            pl.BlockSpec((H, H), lambda j: (0, 0)),
            pl.BlockSpec((1, H), lambda j: (0, 0)),
            pl.BlockSpec((H, H), lambda j: (0, 0)),
        ],
        out_specs=[
            pl.BlockSpec((R, H), lambda j: (j, 0)),
            pl.BlockSpec((R, 1), lambda j: (j, 0)),
        ],
        out_shape=[
            jax.ShapeDtypeStruct((NP, H), jnp.float32),
            jax.ShapeDtypeStruct((NP, 1), jnp.float32),
        ],
    )(x_pad, cnt2, W_in, b_in, W0)


def _layer_body(p_ref, gs_ref, dinv_ref, b_ref, W_ref, out_ref):
    dinv = dinv_ref[...]
    h = jnp.maximum(
        (p_ref[0] + p_ref[1] + gs_ref[...]) * dinv + b_ref[...], 0.0)
    out_ref[...] = jnp.dot(h, W_ref[...],
                           preferred_element_type=jnp.float32) * dinv


def _layer(partials, gs, dinv, b_prev, W_next):
    return pl.pallas_call(
        _layer_body,
        grid=(GRID,),
        in_specs=[
            pl.BlockSpec((2, R, H), lambda j: (0, j, 0)),
            pl.BlockSpec((R, H), lambda j: (j, 0)),
            pl.BlockSpec((R, 1), lambda j: (j, 0)),
            pl.BlockSpec((1, H), lambda j: (0, 0)),
            pl.BlockSpec((H, H), lambda j: (0, 0)),
        ],
        out_specs=pl.BlockSpec((R, H), lambda j: (j, 0)),
        out_shape=jax.ShapeDtypeStruct((NP, H), jnp.float32),
    )(partials, gs, dinv, b_prev, W_next)


def _pool_body(p_ref, gs_ref, dinv_ref, b_ref, batch_ref, W1_ref, b1_ref,
               W2_ref, b2_ref, out_ref, sums_ref, counts_ref):
    j = pl.program_id(0)

    @pl.when(j == 0)
    def _init():
        sums_ref[...] = jnp.zeros_like(sums_ref)
        counts_ref[...] = jnp.zeros_like(counts_ref)
        out_ref[...] = jnp.zeros_like(out_ref)

    dinv = dinv_ref[...]
    h = jnp.maximum(
        (p_ref[0] + p_ref[1] + gs_ref[...]) * dinv + b_ref[...], 0.0)
    gids = lax.broadcasted_iota(jnp.int32, (1, G), 1)
    onehot = (batch_ref[...] == gids).astype(jnp.float32)
    sums_ref[...] += lax.dot_general(
        onehot, h, (((0,), (0,)), ((), ())),
        preferred_element_type=jnp.float32)
    counts_ref[...] += lax.dot_general(
        onehot, jnp.ones((R, 1), jnp.float32), (((0,), (0,)), ((), ())),
        preferred_element_type=jnp.float32)

    @pl.when(j == GRID - 1)
    def _final():
        gemb = sums_ref[...] / jnp.maximum(counts_ref[...], 1.0)
        o = jnp.maximum(
            jnp.dot(gemb, W1_ref[...], preferred_element_type=jnp.float32)
            + b1_ref[...], 0.0)
        out_ref[...] = jnp.dot(o, W2_ref[...],
                               preferred_element_type=jnp.float32) + b2_ref[...]


def _pool(partials, gs, dinv, b_prev, batch2d, W1, b1, W2p, b2p):
    return pl.pallas_call(
        _pool_body,
        grid=(GRID,),
        in_specs=[
            pl.BlockSpec((2, R, H), lambda j: (0, j, 0)),
            pl.BlockSpec((R, H), lambda j: (j, 0)),
            pl.BlockSpec((R, 1), lambda j: (j, 0)),
            pl.BlockSpec((1, H), lambda j: (0, 0)),
            pl.BlockSpec((R, 1), lambda j: (j, 0)),
            pl.BlockSpec((H, H // 2), lambda j: (0, 0)),
            pl.BlockSpec((1, H // 2), lambda j: (0, 0)),
            pl.BlockSpec((H // 2, 16), lambda j: (0, 0)),
            pl.BlockSpec((1, 16), lambda j: (0, 0)),
        ],
        out_specs=pl.BlockSpec((G, 16), lambda j: (0, 0)),
        out_shape=jax.ShapeDtypeStruct((G, 16), jnp.float32),
        scratch_shapes=[
            pltpu.VMEM((G, H), jnp.float32),
            pltpu.VMEM((G, 1), jnp.float32),
        ],
    )(partials, gs, dinv, b_prev, batch2d, W1, b1, W2p, b2p)


def kernel(x, edge_index, batch, W_in, b_in, Ws, bs, gammas, betas,
           bn_means, bn_vars, W1, b1, W2, b2):
    src, dst = edge_index[0], edge_index[1]

    scale = gammas * lax.rsqrt(bn_vars + 1e-5)
    Wp = Ws * scale[:, None, :]
    bp = (bs - bn_means) * scale + betas

    x_pad = jnp.zeros((NP, H), jnp.float32).at[:N].set(x)
    src_pad = jnp.concatenate(
        [src, jnp.zeros((EP - E,), jnp.int32)])
    dst_pad = jnp.concatenate(
        [dst, jnp.full((EP - E,), N, jnp.int32)])
    batch_pad = jnp.concatenate(
        [batch, jnp.full((NP - N,), G, jnp.int32)]).reshape(NP, 1)

    ones128 = jnp.ones((K,), jnp.float32)
    zeros640 = jnp.zeros((ROWS_PER_TILE,), jnp.float32)
    zeros_rows = jnp.zeros((K, H), jnp.float32)
    b_in2 = b_in.reshape(1, H)
    b12 = b1.reshape(1, H // 2)
    W2p = jnp.zeros((H // 2, 16), jnp.float32).at[:, :C].set(W2)
    b2p = jnp.zeros((1, 16), jnp.float32).at[0, :C].set(b2)

    cnt = _degrees(dst_pad, ones128, zeros640)
    cnt2 = cnt.reshape(2, NP, 1)
    gs, dinv = _proj(x_pad, cnt2, W_in, b_in2, Wp[0])

    for i in range(L - 1):
        partials = _message_pass(gs, src_pad, dst_pad, zeros_rows)
        gs = _layer(partials, gs, dinv, bp[i].reshape(1, H), Wp[i + 1])
    partials = _message_pass(gs, src_pad, dst_pad, zeros_rows)

    out16 = _pool(partials, gs, dinv, bp[L - 1].reshape(1, H), batch_pad,
                  W1, b12, W2p, b2p)
    return (out16[:, :C], jnp.float32(0.0))
